# baseline (device time: 52221 ns/iter reference)
import jax
import jax.numpy as jnp
from jax import lax
from jax.experimental import pallas as pl
from jax.experimental.pallas import tpu as pltpu

N_DEV = 32
B = 2
SQ = 128
DMODEL = 512
H_LOC = 4
DH = 64
ROWS = B * SQ
CHUNK = ROWS // N_DEV


def kernel(x, Wq, K_ext, V_ext, Wo):
    def body(x_ref, wq_ref, k_hbm, v_hbm, wo_ref, out_ref,
             partial_ref, inbox_ref, acc_ref, k_ref, v_ref,
             send1, recv1, send2, recv2, sem_k, sem_v):
        me = lax.axis_index("i")

        kcp = pltpu.make_async_copy(
            k_hbm.at[:, :, pl.ds(me * H_LOC, H_LOC), :], k_ref, sem_k)
        vcp = pltpu.make_async_copy(
            v_hbm.at[:, :, pl.ds(me * H_LOC, H_LOC), :], v_ref, sem_v)
        kcp.start()
        vcp.start()

        xq = x_ref[...].reshape(ROWS, DMODEL)
        q_all = jnp.dot(xq, wq_ref[...], preferred_element_type=jnp.float32)
        kcp.wait()
        vcp.wait()
        ctx_rows = []
        for b in range(B):
            head_cols = []
            for h in range(H_LOC):
                q = q_all[b * SQ:(b + 1) * SQ, h * DH:(h + 1) * DH]
                k = k_ref[b, :, h, :]
                v = v_ref[b, :, h, :]
                s = lax.dot_general(
                    q, k, (((1,), (1,)), ((), ())),
                    preferred_element_type=jnp.float32,
                ) * 0.125
                s = s - jnp.max(s, axis=-1, keepdims=True)
                w = jnp.exp(s)
                w = w / jnp.sum(w, axis=-1, keepdims=True)
                head_cols.append(
                    jnp.dot(w, v, preferred_element_type=jnp.float32))
            ctx_rows.append(jnp.concatenate(head_cols, axis=1))
        ctx = jnp.concatenate(ctx_rows, axis=0)
        partial = jnp.dot(ctx, wo_ref[...],
                          preferred_element_type=jnp.float32)
        partial_ref[...] = partial.reshape(N_DEV, CHUNK, DMODEL)
        inbox_ref[0:1] = partial_ref[pl.ds(me, 1)]

        sends1 = []
        for d in range(1, N_DEV):
            peer = (me + d) % N_DEV
            slot = N_DEV - d
            rdma = pltpu.make_async_remote_copy(
                src_ref=partial_ref.at[pl.ds(peer, 1)],
                dst_ref=inbox_ref.at[pl.ds(slot, 1)],
                send_sem=send1.at[d],
                recv_sem=recv1.at[slot],
                device_id=(peer,),
                device_id_type=pl.DeviceIdType.MESH,
            )
            rdma.start()
            sends1.append(rdma)
        for d in range(1, N_DEV):
            pltpu.make_async_remote_copy(
                src_ref=inbox_ref.at[pl.ds(d, 1)],
                dst_ref=inbox_ref.at[pl.ds(d, 1)],
                send_sem=send1.at[d],
                recv_sem=recv1.at[d],
                device_id=(me,),
                device_id_type=pl.DeviceIdType.MESH,
            ).wait_recv()
        for rdma in sends1:
            rdma.wait_send()

        acc_ref[...] = jnp.sum(inbox_ref[...], axis=0, keepdims=True)

        out_ref[pl.ds(me, 1)] = acc_ref[...]
        sends2 = []
        for d in range(1, N_DEV):
            peer = (me + d) % N_DEV
            slot = N_DEV - d
            rdma = pltpu.make_async_remote_copy(
                src_ref=acc_ref,
                dst_ref=out_ref.at[pl.ds(me, 1)],
                send_sem=send2.at[d],
                recv_sem=recv2.at[slot],
                device_id=(peer,),
                device_id_type=pl.DeviceIdType.MESH,
            )
            rdma.start()
            sends2.append(rdma)
        for d in range(1, N_DEV):
            pltpu.make_async_remote_copy(
                src_ref=out_ref.at[pl.ds(d, 1)],
                dst_ref=out_ref.at[pl.ds(d, 1)],
                send_sem=send2.at[d],
                recv_sem=recv2.at[d],
                device_id=(me,),
                device_id_type=pl.DeviceIdType.MESH,
            ).wait_recv()
        for rdma in sends2:
            rdma.wait_send()

    out = pl.pallas_call(
        body,
        out_shape=jax.ShapeDtypeStruct((N_DEV, CHUNK, DMODEL), jnp.float32),
        in_specs=[
            pl.BlockSpec(memory_space=pltpu.VMEM),
            pl.BlockSpec(memory_space=pltpu.VMEM),
            pl.BlockSpec(memory_space=pl.ANY),
            pl.BlockSpec(memory_space=pl.ANY),
            pl.BlockSpec(memory_space=pltpu.VMEM),
        ],
        out_specs=pl.BlockSpec(memory_space=pltpu.VMEM),
        scratch_shapes=[
            pltpu.VMEM((N_DEV, CHUNK, DMODEL), jnp.float32),
            pltpu.VMEM((N_DEV, CHUNK, DMODEL), jnp.float32),
            pltpu.VMEM((1, CHUNK, DMODEL), jnp.float32),
            pltpu.VMEM((B, SQ, H_LOC, DH), jnp.float32),
            pltpu.VMEM((B, SQ, H_LOC, DH), jnp.float32),
            pltpu.SemaphoreType.DMA((N_DEV,)),
            pltpu.SemaphoreType.DMA((N_DEV,)),
            pltpu.SemaphoreType.DMA((N_DEV,)),
            pltpu.SemaphoreType.DMA((N_DEV,)),
            pltpu.SemaphoreType.DMA,
            pltpu.SemaphoreType.DMA,
        ],
    )(x, Wq, K_ext, V_ext, Wo)
    return out.reshape(B, SQ, DMODEL)


# device time: 25875 ns/iter; 2.0182x vs baseline; 2.0182x over previous
import jax
import jax.numpy as jnp
from jax import lax
from jax.experimental import pallas as pl
from jax.experimental.pallas import tpu as pltpu

N_DEV = 32
B = 2
SQ = 128
DMODEL = 512
H_LOC = 4
DH = 64
ROWS = B * SQ
CHUNK = ROWS // N_DEV
CPB = SQ // CHUNK


def kernel(x, Wq, K_ext, V_ext, Wo):
    me_out = lax.axis_index("i")
    sel = lax.dynamic_slice_in_dim(
        jnp.eye(128, dtype=jnp.float32), me_out * H_LOC, H_LOC, axis=1)
    k_loc = jnp.einsum('bjhd,hl->bjld', K_ext, sel)
    v_loc = jnp.einsum('bjhd,hl->bjld', V_ext, sel)

    def body(x_ref, wq_ref, k_ref, v_ref, wo_ref, out_ref,
             partial_ref, inbox_ref, inbox2_ref, acc_ref,
             send1, recv1, send2, recv2):
        me = lax.axis_index("i")

        barrier_sem = pltpu.get_barrier_semaphore()
        for d in range(1, N_DEV):
            pl.semaphore_signal(
                barrier_sem, inc=1,
                device_id=((me + d) % N_DEV,),
                device_id_type=pl.DeviceIdType.MESH,
            )

        xq = x_ref[...].reshape(ROWS, DMODEL)
        q_all = jnp.dot(xq, wq_ref[...], preferred_element_type=jnp.float32)

        def attend(b):
            head_cols = []
            for h in range(H_LOC):
                q = q_all[b * SQ:(b + 1) * SQ, h * DH:(h + 1) * DH]
                k = k_ref[b, :, h, :]
                v = v_ref[b, :, h, :]
                s = lax.dot_general(
                    q, k, (((1,), (1,)), ((), ())),
                    preferred_element_type=jnp.float32,
                ) * 0.125
                s = s - jnp.max(s, axis=-1, keepdims=True)
                w = jnp.exp(s)
                w = w / jnp.sum(w, axis=-1, keepdims=True)
                head_cols.append(
                    jnp.dot(w, v, preferred_element_type=jnp.float32))
            return jnp.dot(jnp.concatenate(head_cols, axis=1), wo_ref[...],
                           preferred_element_type=jnp.float32)

        sends1 = []

        def scatter(c_range):
            for c in c_range:
                rdma = pltpu.make_async_remote_copy(
                    src_ref=partial_ref.at[pl.ds(c, 1)],
                    dst_ref=inbox_ref.at[pl.ds(me, 1)],
                    send_sem=send1.at[c],
                    recv_sem=recv1.at[me],
                    device_id=(c,),
                    device_id_type=pl.DeviceIdType.MESH,
                )

                @pl.when(c != me)
                def _(rdma=rdma):
                    rdma.start()

                sends1.append((c, rdma))

        p0 = attend(0).astype(jnp.bfloat16)
        partial_ref[0:CPB] = p0.reshape(CPB, CHUNK, DMODEL)
        pl.semaphore_wait(barrier_sem, N_DEV - 1)
        scatter(range(CPB))
        p1 = attend(1).astype(jnp.bfloat16)
        partial_ref[CPB:N_DEV] = p1.reshape(CPB, CHUNK, DMODEL)
        scatter(range(CPB, N_DEV))

        inbox_ref[pl.ds(me, 1)] = partial_ref[pl.ds(me, 1)]

        for s in range(N_DEV):
            @pl.when(s != me)
            def _():
                pltpu.make_async_remote_copy(
                    src_ref=inbox_ref.at[pl.ds(s, 1)],
                    dst_ref=inbox_ref.at[pl.ds(s, 1)],
                    send_sem=send1.at[s],
                    recv_sem=recv1.at[s],
                    device_id=(me,),
                    device_id_type=pl.DeviceIdType.MESH,
                ).wait_recv()
        for c, rdma in sends1:
            @pl.when(c != me)
            def _(rdma=rdma):
                rdma.wait_send()

        acc_ref[...] = jnp.sum(
            inbox_ref[...].astype(jnp.float32), axis=0, keepdims=True
        ).astype(jnp.bfloat16)

        inbox2_ref[pl.ds(me, 1)] = acc_ref[...]
        sends2 = []
        for c in range(N_DEV):
            rdma = pltpu.make_async_remote_copy(
                src_ref=acc_ref,
                dst_ref=inbox2_ref.at[pl.ds(me, 1)],
                send_sem=send2.at[c],
                recv_sem=recv2.at[me],
                device_id=(c,),
                device_id_type=pl.DeviceIdType.MESH,
            )

            @pl.when(c != me)
            def _(rdma=rdma):
                rdma.start()

            sends2.append((c, rdma))
        for s in range(N_DEV):
            @pl.when(s != me)
            def _():
                pltpu.make_async_remote_copy(
                    src_ref=inbox2_ref.at[pl.ds(s, 1)],
                    dst_ref=inbox2_ref.at[pl.ds(s, 1)],
                    send_sem=send2.at[s],
                    recv_sem=recv2.at[s],
                    device_id=(me,),
                    device_id_type=pl.DeviceIdType.MESH,
                ).wait_recv()
        out_ref[...] = inbox2_ref[...].astype(jnp.float32)
        for c, rdma in sends2:
            @pl.when(c != me)
            def _(rdma=rdma):
                rdma.wait_send()

    out = pl.pallas_call(
        body,
        out_shape=jax.ShapeDtypeStruct((N_DEV, CHUNK, DMODEL), jnp.float32),
        in_specs=[pl.BlockSpec(memory_space=pltpu.VMEM)] * 5,
        out_specs=pl.BlockSpec(memory_space=pltpu.VMEM),
        scratch_shapes=[
            pltpu.VMEM((N_DEV, CHUNK, DMODEL), jnp.bfloat16),
            pltpu.VMEM((N_DEV, CHUNK, DMODEL), jnp.bfloat16),
            pltpu.VMEM((N_DEV, CHUNK, DMODEL), jnp.bfloat16),
            pltpu.VMEM((1, CHUNK, DMODEL), jnp.bfloat16),
            pltpu.SemaphoreType.DMA((N_DEV,)),
            pltpu.SemaphoreType.DMA((N_DEV,)),
            pltpu.SemaphoreType.DMA((N_DEV,)),
            pltpu.SemaphoreType.DMA((N_DEV,)),
        ],
        compiler_params=pltpu.CompilerParams(collective_id=0),
    )(x, Wq, k_loc, v_loc, Wo)
    return out.reshape(B, SQ, DMODEL)


# device time: 25866 ns/iter; 2.0189x vs baseline; 1.0003x over previous
import jax
import jax.numpy as jnp
from jax import lax
from jax.experimental import pallas as pl
from jax.experimental.pallas import tpu as pltpu

N_DEV = 32
B = 2
SQ = 128
DMODEL = 512
H_LOC = 4
DH = 64
ROWS = B * SQ
CHUNK = ROWS // N_DEV
CPB = SQ // CHUNK


def kernel(x, Wq, K_ext, V_ext, Wo):
    me_out = lax.axis_index("i")
    sel = lax.dynamic_slice_in_dim(
        jnp.eye(128, dtype=jnp.float32), me_out * H_LOC, H_LOC, axis=1)
    k_loc = jnp.einsum('bjhd,hl->bjld', K_ext, sel)
    v_loc = jnp.einsum('bjhd,hl->bjld', V_ext, sel)

    def body(x_ref, wq_ref, k_ref, v_ref, wo_ref, out_ref,
             partial_ref, inbox_ref, inbox2_ref, acc_ref,
             send1, recv1, send2, recv2):
        me = lax.axis_index("i")

        barrier_sem = pltpu.get_barrier_semaphore()
        for d in range(1, N_DEV):
            pl.semaphore_signal(
                barrier_sem, inc=1,
                device_id=((me + d) % N_DEV,),
                device_id_type=pl.DeviceIdType.MESH,
            )

        xq = x_ref[...].reshape(ROWS, DMODEL)
        q_all = jnp.dot(xq, wq_ref[...], preferred_element_type=jnp.float32)

        def attend(b):
            head_cols = []
            for h in range(H_LOC):
                q = q_all[b * SQ:(b + 1) * SQ, h * DH:(h + 1) * DH]
                k = k_ref[b, :, h, :]
                v = v_ref[b, :, h, :]
                s = lax.dot_general(
                    q, k, (((1,), (1,)), ((), ())),
                    preferred_element_type=jnp.float32,
                ) * 0.125
                s = s - jnp.max(s, axis=-1, keepdims=True)
                w = jnp.exp(s)
                w = w / jnp.sum(w, axis=-1, keepdims=True)
                head_cols.append(
                    jnp.dot(w, v, preferred_element_type=jnp.float32))
            return jnp.dot(jnp.concatenate(head_cols, axis=1), wo_ref[...],
                           preferred_element_type=jnp.float32)

        sends1 = []

        def scatter(c_range):
            for c in c_range:
                rdma = pltpu.make_async_remote_copy(
                    src_ref=partial_ref.at[pl.ds(c, 1)],
                    dst_ref=inbox_ref.at[pl.ds(me, 1)],
                    send_sem=send1.at[c],
                    recv_sem=recv1.at[me],
                    device_id=(c,),
                    device_id_type=pl.DeviceIdType.MESH,
                )

                @pl.when(c != me)
                def _(rdma=rdma):
                    rdma.start()

                sends1.append((c, rdma))

        p0 = attend(0).astype(jnp.bfloat16)
        partial_ref[0:CPB] = p0.reshape(CPB, CHUNK, DMODEL)
        pl.semaphore_wait(barrier_sem, N_DEV - 1)
        scatter(range(CPB))
        p1 = attend(1).astype(jnp.bfloat16)
        partial_ref[CPB:N_DEV] = p1.reshape(CPB, CHUNK, DMODEL)
        scatter(range(CPB, N_DEV))

        inbox_ref[pl.ds(me, 1)] = partial_ref[pl.ds(me, 1)]

        for s in range(N_DEV):
            @pl.when(s != me)
            def _():
                pltpu.make_async_remote_copy(
                    src_ref=inbox_ref.at[pl.ds(s, 1)],
                    dst_ref=inbox_ref.at[pl.ds(s, 1)],
                    send_sem=send1.at[s],
                    recv_sem=recv1.at[s],
                    device_id=(me,),
                    device_id_type=pl.DeviceIdType.MESH,
                ).wait_recv()
        for c, rdma in sends1:
            @pl.when(c != me)
            def _(rdma=rdma):
                rdma.wait_send()

        acc_ref[...] = jnp.sum(
            inbox_ref[...].astype(jnp.float32), axis=0, keepdims=True
        ).astype(jnp.bfloat16)

        inbox2_ref[pl.ds(me, 1)] = acc_ref[...]
        sends2 = []
        for c in range(N_DEV):
            rdma = pltpu.make_async_remote_copy(
                src_ref=acc_ref,
                dst_ref=inbox2_ref.at[pl.ds(me, 1)],
                send_sem=send2.at[c],
                recv_sem=recv2.at[me],
                device_id=(c,),
                device_id_type=pl.DeviceIdType.MESH,
            )

            @pl.when(c != me)
            def _(rdma=rdma):
                rdma.start()

            sends2.append((c, rdma))
        for s in range(N_DEV):
            @pl.when(s != me)
            def _():
                pltpu.make_async_remote_copy(
                    src_ref=inbox2_ref.at[pl.ds(s, 1)],
                    dst_ref=inbox2_ref.at[pl.ds(s, 1)],
                    send_sem=send2.at[s],
                    recv_sem=recv2.at[s],
                    device_id=(me,),
                    device_id_type=pl.DeviceIdType.MESH,
                ).wait_recv()
        out_ref[...] = inbox2_ref[...].astype(jnp.float32).reshape(
            B, SQ, DMODEL)
        for c, rdma in sends2:
            @pl.when(c != me)
            def _(rdma=rdma):
                rdma.wait_send()

    return pl.pallas_call(
        body,
        out_shape=jax.ShapeDtypeStruct((B, SQ, DMODEL), jnp.float32),
        in_specs=[pl.BlockSpec(memory_space=pltpu.VMEM)] * 5,
        out_specs=pl.BlockSpec(memory_space=pltpu.VMEM),
        scratch_shapes=[
            pltpu.VMEM((N_DEV, CHUNK, DMODEL), jnp.bfloat16),
            pltpu.VMEM((N_DEV, CHUNK, DMODEL), jnp.bfloat16),
            pltpu.VMEM((N_DEV, CHUNK, DMODEL), jnp.bfloat16),
            pltpu.VMEM((1, CHUNK, DMODEL), jnp.bfloat16),
            pltpu.SemaphoreType.DMA((N_DEV,)),
            pltpu.SemaphoreType.DMA((N_DEV,)),
            pltpu.SemaphoreType.DMA((N_DEV,)),
            pltpu.SemaphoreType.DMA((N_DEV,)),
        ],
        compiler_params=pltpu.CompilerParams(collective_id=0),
    )(x, Wq, k_loc, v_loc, Wo)
